# Initial kernel scaffold; baseline (speedup 1.0000x reference)
import functools

import jax
import jax.numpy as jnp
from jax import lax
from jax.experimental import pallas as pl
from jax.experimental.pallas import tpu as pltpu
from jax.experimental.pallas import tpu_sc as plsc

N = 10000
E = 320000
D = 128
NC = 2
NS = 16
NW = NC * NS
EPW = E // NW
CH = 80
NCHUNK = EPW // CH
RPT = N // NS

_mesh = plsc.VectorSubcoreMesh(core_axis_name="c", subcore_axis_name="s")


def _zero_shared(zbuf, accum, sid, rows_per_copy, ncopies, lanes):
    nvec = lanes // 16

    def zbody(i, _):
        for j in range(nvec):
            zbuf[i, pl.ds(16 * j, 16)] = jnp.zeros((16,), jnp.float32)
        return 0

    lax.fori_loop(0, rows_per_copy, zbody, 0)
    for r in range(ncopies):
        pltpu.sync_copy(zbuf, accum.at[pl.ds(sid * RPT + r * rows_per_copy,
                                             rows_per_copy)])


@functools.partial(
    pl.kernel,
    out_type=jax.ShapeDtypeStruct((NC, N, 16), jnp.float32),
    mesh=_mesh,
    scratch_types=[
        pltpu.VMEM_SHARED((N, 16), jnp.float32),
        pltpu.VMEM((RPT, 16), jnp.float32),
        pltpu.VMEM((CH,), jnp.int32),
        pltpu.VMEM((CH,), jnp.float32),
        pltpu.VMEM((CH, 16), jnp.float32),
    ],
)
def _deg_kernel(dst_hbm, ew_hbm, out_hbm, accum, zbuf, dsti, ewv, ewb):
    cid = lax.axis_index("c")
    sid = lax.axis_index("s")
    wid = sid * NC + cid

    _zero_shared(zbuf, accum, sid, RPT, 1, 16)
    plsc.subcore_barrier()

    def chunk(c, _):
        base = wid * EPW + c * CH
        pltpu.sync_copy(dst_hbm.at[pl.ds(base, CH)], dsti)
        pltpu.sync_copy(ew_hbm.at[pl.ds(base, CH)], ewv)

        def bcast(k, _):
            s = ewv[k]
            ewb[k, :] = jnp.full((16,), s, jnp.float32)
            return 0

        lax.fori_loop(0, CH, bcast, 0)
        pltpu.sync_copy(ewb, accum.at[dsti], add=True)
        return 0

    lax.fori_loop(0, NCHUNK, chunk, 0)
    plsc.subcore_barrier()
    pltpu.sync_copy(accum.at[pl.ds(sid * RPT, RPT)],
                    out_hbm.at[cid, pl.ds(sid * RPT, RPT)])


@functools.partial(
    pl.kernel,
    out_type=jax.ShapeDtypeStruct((NC, N, D), jnp.float32),
    mesh=_mesh,
    scratch_types=[
        pltpu.VMEM_SHARED((N, D), jnp.float32),
        pltpu.VMEM((RPT // 5, D), jnp.float32),
        pltpu.VMEM((CH,), jnp.int32),
        pltpu.VMEM((CH,), jnp.int32),
        pltpu.VMEM((CH,), jnp.float32),
        pltpu.VMEM((CH, D), jnp.float32),
        pltpu.SemaphoreType.DMA,
    ],
)
def _scatter_kernel(g_hbm, src_hbm, dst_hbm, ew_hbm, out_hbm,
                    accum, zbuf, srci, dsti, ewv, rows, sem):
    cid = lax.axis_index("c")
    sid = lax.axis_index("s")
    wid = sid * NC + cid

    _zero_shared(zbuf, accum, sid, RPT // 5, 5, D)
    plsc.subcore_barrier()

    def chunk(c, _):
        base = wid * EPW + c * CH
        pltpu.sync_copy(src_hbm.at[pl.ds(base, CH)], srci)
        pltpu.sync_copy(dst_hbm.at[pl.ds(base, CH)], dsti)
        pltpu.sync_copy(ew_hbm.at[pl.ds(base, CH)], ewv)
        pltpu.async_copy(g_hbm.at[srci], rows, sem).wait()

        def scale(k, _):
            s = ewv[k]
            for j in range(D // 16):
                sl = pl.ds(16 * j, 16)
                rows[k, sl] = rows[k, sl] * s
            return 0

        lax.fori_loop(0, CH, scale, 0)
        pltpu.sync_copy(rows, accum.at[dsti], add=True)
        return 0

    lax.fori_loop(0, NCHUNK, chunk, 0)
    plsc.subcore_barrier()
    pltpu.sync_copy(accum.at[pl.ds(sid * RPT, RPT)],
                    out_hbm.at[cid, pl.ds(sid * RPT, RPT)])


_BR = 500
_GRID = N // _BR


def _dinv_block(deg0, deg1):
    deg = deg0[:, 0:1] + deg1[:, 0:1] + 1.0
    return jnp.where(deg > 0, lax.rsqrt(jnp.maximum(deg, 1e-12)), 0.0)


def _tc1_body(deg0, deg1, x, w1, g1):
    dinv = _dinv_block(deg0, deg1)
    g1[:, :] = dinv * jnp.dot(x[:, :], w1[:, :],
                              preferred_element_type=jnp.float32)


def _tc2_body(deg0, deg1, s1a, s1b, g1, b1, w2, g2):
    dinv = _dinv_block(deg0, deg1)
    h1 = jnp.maximum(dinv * (s1a[:, :] + s1b[:, :] + g1[:, :]) + b1[:, :], 0.0)
    g2[:, :] = dinv * jnp.dot(h1, w2[:, :], preferred_element_type=jnp.float32)


def _tc3_body(deg0, deg1, s2a, s2b, g2, b2, out):
    dinv = _dinv_block(deg0, deg1)
    out[:, :] = jnp.maximum(dinv * (s2a[:, :] + s2b[:, :] + g2[:, :])
                            + b2[:, :], 0.0)


def _row_spec():
    return pl.BlockSpec((_BR, D), lambda i: (i, 0))


def _deg_spec():
    return pl.BlockSpec((_BR, 16), lambda i: (i, 0))


def _full_spec(shape):
    return pl.BlockSpec(shape, lambda i: tuple(0 for _ in shape))


def _tc1(deg0, deg1, x, w1):
    return pl.pallas_call(
        _tc1_body,
        grid=(_GRID,),
        in_specs=[_deg_spec(), _deg_spec(), _row_spec(), _full_spec((D, D))],
        out_specs=_row_spec(),
        out_shape=jax.ShapeDtypeStruct((N, D), jnp.float32),
    )(deg0, deg1, x, w1)


def _tc2(deg0, deg1, s1a, s1b, g1, b1, w2):
    return pl.pallas_call(
        _tc2_body,
        grid=(_GRID,),
        in_specs=[_deg_spec(), _deg_spec(), _row_spec(), _row_spec(),
                  _row_spec(), _full_spec((1, D)), _full_spec((D, D))],
        out_specs=_row_spec(),
        out_shape=jax.ShapeDtypeStruct((N, D), jnp.float32),
    )(deg0, deg1, s1a, s1b, g1, b1, w2)


def _tc3(deg0, deg1, s2a, s2b, g2, b2):
    return pl.pallas_call(
        _tc3_body,
        grid=(_GRID,),
        in_specs=[_deg_spec(), _deg_spec(), _row_spec(), _row_spec(),
                  _row_spec(), _full_spec((1, D))],
        out_specs=_row_spec(),
        out_shape=jax.ShapeDtypeStruct((N, D), jnp.float32),
    )(deg0, deg1, s2a, s2b, g2, b2)


def kernel(x, edge_index, edge_weight, W1, b1, W2, b2):
    src = edge_index[0].astype(jnp.int32)
    dst = edge_index[1].astype(jnp.int32)
    ew = edge_weight.astype(jnp.float32)

    deg_parts = _deg_kernel(dst, ew)
    deg0, deg1 = deg_parts[0], deg_parts[1]

    g1 = _tc1(deg0, deg1, x, W1)
    s1 = _scatter_kernel(g1, src, dst, ew)
    g2 = _tc2(deg0, deg1, s1[0], s1[1], g1, b1.reshape(1, D), W2)
    s2 = _scatter_kernel(g2, src, dst, ew)
    out = _tc3(deg0, deg1, s2[0], s2[1], g2, b2.reshape(1, D))
    return out


# speedup vs baseline: 8.8954x; 8.8954x over previous
import functools

import jax
import jax.numpy as jnp
from jax import lax
from jax.experimental import pallas as pl
from jax.experimental.pallas import tpu as pltpu
from jax.experimental.pallas import tpu_sc as plsc

N = 10000
E = 320000
D = 128
NC = 2
NS = 16
NW = NC * NS
EPW = E // NW
CH = 80
NCHUNK = EPW // CH
RPT = N // NS

_mesh = plsc.VectorSubcoreMesh(core_axis_name="c", subcore_axis_name="s")


def _zero_shared(zbuf, accum, sid, rows_per_copy, ncopies, lanes):
    nvec = lanes // 16

    def zbody(i, _):
        for j in range(nvec):
            zbuf[i, pl.ds(16 * j, 16)] = jnp.zeros((16,), jnp.float32)
        return 0

    lax.fori_loop(0, rows_per_copy, zbody, 0)
    for r in range(ncopies):
        pltpu.sync_copy(zbuf, accum.at[pl.ds(sid * RPT + r * rows_per_copy,
                                             rows_per_copy)])


NP = 10240
_DSEG = NP // 8
_DEG_KW = dict(
    out_type=jax.ShapeDtypeStruct((NC, NP), jnp.float32),
    mesh=_mesh,
    scratch_types=[
        pltpu.VMEM((NP,), jnp.float32),
        pltpu.VMEM_SHARED((NS, NP), jnp.float32),
        pltpu.VMEM((NS, _DSEG), jnp.float32),
        pltpu.VMEM((_DSEG,), jnp.float32),
        pltpu.VMEM((CH,), jnp.int32),
        pltpu.VMEM((CH,), jnp.float32),
    ],
)


def _deg_body(dst_hbm, ew_hbm, out_hbm, degloc, stag, rbuf, red, dsti, ewv):
    cid = lax.axis_index("c")
    sid = lax.axis_index("s")
    wid = sid * NC + cid

    def zbody(i, _):
        degloc[pl.ds(i * 16, 16)] = jnp.zeros((16,), jnp.float32)
        return 0

    lax.fori_loop(0, NP // 16, zbody, 0)

    iota16 = lax.iota(jnp.int32, 16)

    def chunk(c, _):
        base = wid * EPW + c * CH
        pltpu.sync_copy(dst_hbm.at[pl.ds(base, CH)], dsti)
        pltpu.sync_copy(ew_hbm.at[pl.ds(base, CH)], ewv)
        for m in range(CH // 16):
            dvec = dsti[pl.ds(m * 16, 16)]
            wvec = ewv[pl.ds(m * 16, 16)]
            for l in range(16):
                d = dvec[l]
                rbase = (d >> 4) * 16
                lane = d - rbase
                sl = pl.ds(rbase, 16)
                degloc[sl] = degloc[sl] + jnp.where(iota16 == lane,
                                                    wvec[l], 0.0)
        return 0

    lax.fori_loop(0, NCHUNK, chunk, 0)

    pltpu.sync_copy(degloc, stag.at[sid])
    plsc.subcore_barrier()

    @pl.when(sid < NP // _DSEG)
    def _():
        pltpu.sync_copy(stag.at[:, pl.ds(sid * _DSEG, _DSEG)], rbuf)

        def rb(v, _):
            sl = pl.ds(v * 16, 16)
            acc = rbuf[0, sl]
            for r in range(1, NS):
                acc = acc + rbuf[r, sl]
            red[sl] = acc
            return 0

        lax.fori_loop(0, _DSEG // 16, rb, 0)
        pltpu.sync_copy(red, out_hbm.at[cid, pl.ds(sid * _DSEG, _DSEG)])


_deg_kernel = functools.partial(pl.kernel, **_DEG_KW)(_deg_body)


_SCAT_KW = dict(
    out_type=jax.ShapeDtypeStruct((NC, NS, RPT, D), jnp.float32),
    mesh=_mesh,
    scratch_types=[
        pltpu.VMEM_SHARED((N, D), jnp.float32),
        pltpu.VMEM((RPT // 5, D), jnp.float32),
        pltpu.VMEM((CH,), jnp.int32),
        pltpu.VMEM((CH,), jnp.int32),
        pltpu.VMEM((CH,), jnp.float32),
        pltpu.VMEM((CH, D), jnp.float32),
        pltpu.SemaphoreType.DMA,
    ],
)


def _scatter_body(g_hbm, src_hbm, dst_hbm, ew_hbm, out_hbm,
                  accum, zbuf, srci, dsti, ewv, rows, sem):
    cid = lax.axis_index("c")
    sid = lax.axis_index("s")
    wid = sid * NC + cid

    _zero_shared(zbuf, accum, sid, RPT // 5, 5, D)
    plsc.subcore_barrier()

    def chunk(c, _):
        base = wid * EPW + c * CH
        pltpu.sync_copy(src_hbm.at[pl.ds(base, CH)], srci)
        pltpu.sync_copy(dst_hbm.at[pl.ds(base, CH)], dsti)
        pltpu.sync_copy(ew_hbm.at[pl.ds(base, CH)], ewv)
        pltpu.async_copy(g_hbm.at[srci], rows, sem).wait()

        def scale(m, _):
            wvec = ewv[pl.ds(m * 16, 16)]
            for l in range(16):
                k = m * 16 + l
                s = wvec[l]
                for j in range(D // 16):
                    sl = pl.ds(16 * j, 16)
                    rows[k, sl] = rows[k, sl] * s
            return 0

        lax.fori_loop(0, CH // 16, scale, 0)
        pltpu.sync_copy(rows, accum.at[dsti], add=True)
        return 0

    lax.fori_loop(0, NCHUNK, chunk, 0)
    plsc.subcore_barrier()
    pltpu.sync_copy(accum.at[pl.ds(sid * RPT, RPT)], out_hbm.at[cid, sid])


_scatter_kernel = functools.partial(pl.kernel, **_SCAT_KW)(_scatter_body)


_BR = 400
_GRID = N // _BR


def _dinv_block(deg0, deg1):
    deg = deg0[:, :] + deg1[:, :] + 1.0
    return jnp.where(deg > 0, lax.rsqrt(jnp.maximum(deg, 1e-12)), 0.0)


def _tc1_body(deg0, deg1, x, w1, g1):
    dinv = _dinv_block(deg0, deg1)
    g1[:, :] = dinv * jnp.dot(x[:, :], w1[:, :],
                              preferred_element_type=jnp.float32)


def _tc2_body(deg0, deg1, s1a, s1b, g1, b1, w2, g2):
    dinv = _dinv_block(deg0, deg1)
    h1 = jnp.maximum(dinv * (s1a[:, :] + s1b[:, :] + g1[:, :]) + b1[:, :], 0.0)
    g2[:, :] = dinv * jnp.dot(h1, w2[:, :], preferred_element_type=jnp.float32)


def _tc3_body(deg0, deg1, s2a, s2b, g2, b2, out):
    dinv = _dinv_block(deg0, deg1)
    out[:, :] = jnp.maximum(dinv * (s2a[:, :] + s2b[:, :] + g2[:, :])
                            + b2[:, :], 0.0)


def _row_spec():
    return pl.BlockSpec((_BR, D), lambda i: (i, 0))


def _deg_spec():
    return pl.BlockSpec((_BR, 1), lambda i: (i, 0))


def _full_spec(shape):
    return pl.BlockSpec(shape, lambda i: tuple(0 for _ in shape))


def _tc1(deg0, deg1, x, w1):
    return pl.pallas_call(
        _tc1_body,
        grid=(_GRID,),
        in_specs=[_deg_spec(), _deg_spec(), _row_spec(), _full_spec((D, D))],
        out_specs=_row_spec(),
        out_shape=jax.ShapeDtypeStruct((N, D), jnp.float32),
    )(deg0, deg1, x, w1)


def _tc2(deg0, deg1, s1a, s1b, g1, b1, w2):
    return pl.pallas_call(
        _tc2_body,
        grid=(_GRID,),
        in_specs=[_deg_spec(), _deg_spec(), _row_spec(), _row_spec(),
                  _row_spec(), _full_spec((1, D)), _full_spec((D, D))],
        out_specs=_row_spec(),
        out_shape=jax.ShapeDtypeStruct((N, D), jnp.float32),
    )(deg0, deg1, s1a, s1b, g1, b1, w2)


def _tc3(deg0, deg1, s2a, s2b, g2, b2):
    return pl.pallas_call(
        _tc3_body,
        grid=(_GRID,),
        in_specs=[_deg_spec(), _deg_spec(), _row_spec(), _row_spec(),
                  _row_spec(), _full_spec((1, D))],
        out_specs=_row_spec(),
        out_shape=jax.ShapeDtypeStruct((N, D), jnp.float32),
    )(deg0, deg1, s2a, s2b, g2, b2)


def kernel(x, edge_index, edge_weight, W1, b1, W2, b2):
    src = edge_index[0].astype(jnp.int32)
    dst = edge_index[1].astype(jnp.int32)
    ew = edge_weight.astype(jnp.float32)

    degp = _deg_kernel(dst, ew)
    deg0 = degp[0, :N].reshape(N, 1)
    deg1 = degp[1, :N].reshape(N, 1)

    g1 = _tc1(deg0, deg1, x, W1)
    s1 = _scatter_kernel(g1, src, dst, ew).reshape(NC, N, D)
    g2 = _tc2(deg0, deg1, s1[0], s1[1], g1, b1.reshape(1, D), W2)
    s2 = _scatter_kernel(g2, src, dst, ew).reshape(NC, N, D)
    out = _tc3(deg0, deg1, s2[0], s2[1], g2, b2.reshape(1, D))
    return out


# speedup vs baseline: 17.8231x; 2.0036x over previous
import functools

import jax
import jax.numpy as jnp
from jax import lax
from jax.experimental import pallas as pl
from jax.experimental.pallas import tpu as pltpu
from jax.experimental.pallas import tpu_sc as plsc

N = 10000
E = 320000
D = 128
NC = 2
NS = 16
NW = NC * NS
EPW = E // NW
CH = 80
NCHUNK = EPW // CH
RPT = N // NS

_mesh = plsc.VectorSubcoreMesh(core_axis_name="c", subcore_axis_name="s")


def _zero_shared(zbuf, accum, sid, rows_per_copy, ncopies, lanes):
    nvec = lanes // 16

    def zbody(i, _):
        for j in range(nvec):
            zbuf[i, pl.ds(16 * j, 16)] = jnp.zeros((16,), jnp.float32)
        return 0

    lax.fori_loop(0, rows_per_copy, zbody, 0)
    for r in range(ncopies):
        pltpu.sync_copy(zbuf, accum.at[pl.ds(sid * RPT + r * rows_per_copy,
                                             rows_per_copy)])


NP = 10240
_DSEG = NP // 8
_DEG_KW = dict(
    out_type=jax.ShapeDtypeStruct((NC, NP), jnp.float32),
    mesh=_mesh,
    scratch_types=[
        pltpu.VMEM((NP,), jnp.float32),
        pltpu.VMEM_SHARED((NS, NP), jnp.float32),
        pltpu.VMEM((NS, _DSEG), jnp.float32),
        pltpu.VMEM((_DSEG,), jnp.float32),
        pltpu.VMEM((EPW,), jnp.int32),
        pltpu.VMEM((EPW,), jnp.float32),
    ],
)


def _deg_body(dst_hbm, ew_hbm, out_hbm, degloc, stag, rbuf, red, dsta, ewa):
    cid = lax.axis_index("c")
    sid = lax.axis_index("s")
    wid = sid * NC + cid

    pltpu.sync_copy(dst_hbm.at[pl.ds(wid * EPW, EPW)], dsta)
    pltpu.sync_copy(ew_hbm.at[pl.ds(wid * EPW, EPW)], ewa)

    def zbody(i, _):
        degloc[pl.ds(i * 16, 16)] = jnp.zeros((16,), jnp.float32)
        return 0

    lax.fori_loop(0, NP // 16, zbody, 0)

    iota16 = lax.iota(jnp.int32, 16)

    def chunk(m, _):
        dvec = dsta[pl.ds(m * 16, 16)]
        wvec = ewa[pl.ds(m * 16, 16)]
        for l in range(16):
            d = dvec[l]
            rbase = (d >> 4) * 16
            lane = d - rbase
            sl = pl.ds(rbase, 16)
            degloc[sl] = degloc[sl] + jnp.where(iota16 == lane, wvec[l], 0.0)
        return 0

    lax.fori_loop(0, EPW // 16, chunk, 0)

    pltpu.sync_copy(degloc, stag.at[sid])
    plsc.subcore_barrier()

    @pl.when(sid < NP // _DSEG)
    def _():
        pltpu.sync_copy(stag.at[:, pl.ds(sid * _DSEG, _DSEG)], rbuf)

        def rb(v, _):
            sl = pl.ds(v * 16, 16)
            acc = rbuf[0, sl]
            for r in range(1, NS):
                acc = acc + rbuf[r, sl]
            red[sl] = acc
            return 0

        lax.fori_loop(0, _DSEG // 16, rb, 0)
        pltpu.sync_copy(red, out_hbm.at[cid, pl.ds(sid * _DSEG, _DSEG)])


_deg_kernel = functools.partial(pl.kernel, **_DEG_KW)(_deg_body)


_ZR = 25
_SCAT_KW = dict(
    out_type=jax.ShapeDtypeStruct((NC, NS, RPT, D), jnp.float32),
    mesh=_mesh,
    scratch_types=[
        pltpu.VMEM_SHARED((N, D), jnp.float32),
        pltpu.VMEM((_ZR, D), jnp.float32),
        pltpu.VMEM((EPW,), jnp.int32),
        pltpu.VMEM((EPW,), jnp.float32),
        pltpu.VMEM((CH,), jnp.int32),
        pltpu.VMEM((CH,), jnp.int32),
        pltpu.VMEM((CH,), jnp.int32),
        pltpu.VMEM((CH,), jnp.int32),
        pltpu.VMEM((CH, D), jnp.float32),
        pltpu.VMEM((CH, D), jnp.float32),
        pltpu.SemaphoreType.DMA,
        pltpu.SemaphoreType.DMA,
        pltpu.SemaphoreType.DMA,
        pltpu.SemaphoreType.DMA,
        pltpu.SemaphoreType.DMA,
        pltpu.SemaphoreType.DMA,
    ],
)


def _scatter_body(g_hbm, src_hbm, dst_hbm, ew_hbm, out_hbm,
                  accum, zbuf, srca, ewa, dstg0, dstg1, dsti0, dsti1,
                  rows0, rows1, gsem0, gsem1, dsem0, dsem1, ssem0, ssem1):
    cid = lax.axis_index("c")
    sid = lax.axis_index("s")
    wid = sid * NC + cid
    base = wid * EPW

    pltpu.sync_copy(src_hbm.at[pl.ds(base, EPW)], srca)
    pltpu.sync_copy(ew_hbm.at[pl.ds(base, EPW)], ewa)

    _zero_shared(zbuf, accum, sid, _ZR, RPT // _ZR, D)
    plsc.subcore_barrier()

    def fetch(c, rows, gsem, dstg, dsem):
        pltpu.async_copy(g_hbm.at[srca.at[pl.ds(c * CH, CH)]], rows, gsem)
        pltpu.async_copy(dst_hbm.at[pl.ds(base + c * CH, CH)], dstg, dsem)

    def wait_fetch(c, rows, gsem, dstg, dsem):
        pltpu.make_async_copy(g_hbm.at[srca.at[pl.ds(c * CH, CH)]],
                              rows, gsem).wait()
        pltpu.make_async_copy(dst_hbm.at[pl.ds(base + c * CH, CH)],
                              dstg, dsem).wait()

    def scale(c, rows):
        def body(m, _):
            wvec = ewa[pl.ds(c * CH + m * 16, 16)]
            for l in range(16):
                k = m * 16 + l
                s = wvec[l]
                for j in range(D // 16):
                    sl = pl.ds(16 * j, 16)
                    rows[k, sl] = rows[k, sl] * s
            return 0

        lax.fori_loop(0, CH // 16, body, 0)

    def fill_dsti(dstg, dsti):
        for m in range(CH // 16):
            sl = pl.ds(m * 16, 16)
            dsti[sl] = dstg[sl]

    def add_start(rows, dsti, ssem):
        pltpu.async_copy(rows, accum.at[dsti], ssem, add=True)

    def add_wait(rows, dsti, ssem):
        pltpu.make_async_copy(rows, accum.at[dsti], ssem).wait()

    fetch(0, rows0, gsem0, dstg0, dsem0)

    def pair(g, _):
        c0 = 2 * g
        c1 = c0 + 1
        wait_fetch(c0, rows0, gsem0, dstg0, dsem0)

        @pl.when(g > 0)
        def _():
            add_wait(rows1, dsti1, ssem1)

        fetch(c1, rows1, gsem1, dstg1, dsem1)
        scale(c0, rows0)
        fill_dsti(dstg0, dsti0)
        add_start(rows0, dsti0, ssem0)
        wait_fetch(c1, rows1, gsem1, dstg1, dsem1)
        scale(c1, rows1)
        fill_dsti(dstg1, dsti1)
        add_wait(rows0, dsti0, ssem0)
        fetch(c1 + 1, rows0, gsem0, dstg0, dsem0)
        add_start(rows1, dsti1, ssem1)
        return 0

    lax.fori_loop(0, NCHUNK // 2, pair, 0)
    cl = NCHUNK - 1
    wait_fetch(cl, rows0, gsem0, dstg0, dsem0)
    add_wait(rows1, dsti1, ssem1)
    scale(cl, rows0)
    fill_dsti(dstg0, dsti0)
    add_start(rows0, dsti0, ssem0)
    add_wait(rows0, dsti0, ssem0)

    plsc.subcore_barrier()
    pltpu.sync_copy(accum.at[pl.ds(sid * RPT, RPT)], out_hbm.at[cid, sid])


_scatter_kernel = functools.partial(pl.kernel, **_SCAT_KW)(_scatter_body)


_BR = 400
_GRID = N // _BR


def _dinv_block(deg0, deg1):
    deg = deg0[:, :] + deg1[:, :] + 1.0
    return jnp.where(deg > 0, lax.rsqrt(jnp.maximum(deg, 1e-12)), 0.0)


def _tc1_body(deg0, deg1, x, w1, g1):
    dinv = _dinv_block(deg0, deg1)
    g1[:, :] = dinv * jnp.dot(x[:, :], w1[:, :],
                              preferred_element_type=jnp.float32)


def _tc2_body(deg0, deg1, s1a, s1b, g1, b1, w2, g2):
    dinv = _dinv_block(deg0, deg1)
    h1 = jnp.maximum(dinv * (s1a[:, :] + s1b[:, :] + g1[:, :]) + b1[:, :], 0.0)
    g2[:, :] = dinv * jnp.dot(h1, w2[:, :], preferred_element_type=jnp.float32)


def _tc3_body(deg0, deg1, s2a, s2b, g2, b2, out):
    dinv = _dinv_block(deg0, deg1)
    out[:, :] = jnp.maximum(dinv * (s2a[:, :] + s2b[:, :] + g2[:, :])
                            + b2[:, :], 0.0)


def _row_spec():
    return pl.BlockSpec((_BR, D), lambda i: (i, 0))


def _deg_spec():
    return pl.BlockSpec((_BR, 1), lambda i: (i, 0))


def _full_spec(shape):
    return pl.BlockSpec(shape, lambda i: tuple(0 for _ in shape))


def _tc1(deg0, deg1, x, w1):
    return pl.pallas_call(
        _tc1_body,
        grid=(_GRID,),
        in_specs=[_deg_spec(), _deg_spec(), _row_spec(), _full_spec((D, D))],
        out_specs=_row_spec(),
        out_shape=jax.ShapeDtypeStruct((N, D), jnp.float32),
    )(deg0, deg1, x, w1)


def _tc2(deg0, deg1, s1a, s1b, g1, b1, w2):
    return pl.pallas_call(
        _tc2_body,
        grid=(_GRID,),
        in_specs=[_deg_spec(), _deg_spec(), _row_spec(), _row_spec(),
                  _row_spec(), _full_spec((1, D)), _full_spec((D, D))],
        out_specs=_row_spec(),
        out_shape=jax.ShapeDtypeStruct((N, D), jnp.float32),
    )(deg0, deg1, s1a, s1b, g1, b1, w2)


def _tc3(deg0, deg1, s2a, s2b, g2, b2):
    return pl.pallas_call(
        _tc3_body,
        grid=(_GRID,),
        in_specs=[_deg_spec(), _deg_spec(), _row_spec(), _row_spec(),
                  _row_spec(), _full_spec((1, D))],
        out_specs=_row_spec(),
        out_shape=jax.ShapeDtypeStruct((N, D), jnp.float32),
    )(deg0, deg1, s2a, s2b, g2, b2)


def kernel(x, edge_index, edge_weight, W1, b1, W2, b2):
    src = edge_index[0].astype(jnp.int32)
    dst = edge_index[1].astype(jnp.int32)
    ew = edge_weight.astype(jnp.float32)

    degp = _deg_kernel(dst, ew)
    deg0 = degp[0, :N].reshape(N, 1)
    deg1 = degp[1, :N].reshape(N, 1)

    g1 = _tc1(deg0, deg1, x, W1)
    s1 = _scatter_kernel(g1, src, dst, ew).reshape(NC, N, D)
    g2 = _tc2(deg0, deg1, s1[0], s1[1], g1, b1.reshape(1, D), W2)
    s2 = _scatter_kernel(g2, src, dst, ew).reshape(NC, N, D)
    out = _tc3(deg0, deg1, s2[0], s2[1], g2, b2.reshape(1, D))
    return out


# speedup vs baseline: 18.1010x; 1.0156x over previous
import functools

import jax
import jax.numpy as jnp
from jax import lax
from jax.experimental import pallas as pl
from jax.experimental.pallas import tpu as pltpu
from jax.experimental.pallas import tpu_sc as plsc

N = 10000
E = 320000
D = 128
NC = 2
NS = 16
NW = NC * NS
EPW = E // NW
CH = 80
NCHUNK = EPW // CH
RPT = N // NS

_mesh = plsc.VectorSubcoreMesh(core_axis_name="c", subcore_axis_name="s")


def _zero_shared(zbuf, accum, sid, rows_per_copy, ncopies, lanes, zsem):
    nvec = lanes // 16

    def zbody(i, _):
        for j in range(nvec):
            zbuf[i, pl.ds(16 * j, 16)] = jnp.zeros((16,), jnp.float32)
        return 0

    lax.fori_loop(0, rows_per_copy, zbody, 0)
    for r in range(ncopies):
        pltpu.async_copy(zbuf, accum.at[pl.ds(sid * RPT + r * rows_per_copy,
                                              rows_per_copy)], zsem)
    for r in range(ncopies):
        pltpu.make_async_copy(zbuf, accum.at[pl.ds(sid * RPT + r * rows_per_copy,
                                                   rows_per_copy)], zsem).wait()


NP = 10240
_DSEG = NP // 8
_DEG_KW = dict(
    out_type=jax.ShapeDtypeStruct((NC, NP), jnp.float32),
    mesh=_mesh,
    scratch_types=[
        pltpu.VMEM((NP,), jnp.float32),
        pltpu.VMEM_SHARED((NS, NP), jnp.float32),
        pltpu.VMEM((NS, _DSEG), jnp.float32),
        pltpu.VMEM((_DSEG,), jnp.float32),
        pltpu.VMEM((EPW,), jnp.int32),
        pltpu.VMEM((EPW,), jnp.float32),
        pltpu.SemaphoreType.DMA,
        pltpu.SemaphoreType.DMA,
    ],
)


def _deg_body(dst_hbm, ew_hbm, out_hbm, degloc, stag, rbuf, red, dsta, ewa,
              dgsem0, dgsem1):
    cid = lax.axis_index("c")
    sid = lax.axis_index("s")
    wid = sid * NC + cid

    pltpu.async_copy(dst_hbm.at[pl.ds(wid * EPW, EPW)], dsta, dgsem0)
    pltpu.async_copy(ew_hbm.at[pl.ds(wid * EPW, EPW)], ewa, dgsem1)

    def zbody(i, _):
        degloc[pl.ds(i * 16, 16)] = jnp.zeros((16,), jnp.float32)
        return 0

    lax.fori_loop(0, NP // 16, zbody, 0)
    pltpu.make_async_copy(dst_hbm.at[pl.ds(wid * EPW, EPW)], dsta, dgsem0).wait()
    pltpu.make_async_copy(ew_hbm.at[pl.ds(wid * EPW, EPW)], ewa, dgsem1).wait()

    iota16 = lax.iota(jnp.int32, 16)

    def chunk(m, _):
        dvec = dsta[pl.ds(m * 16, 16)]
        wvec = ewa[pl.ds(m * 16, 16)]
        for l in range(16):
            d = dvec[l]
            rbase = (d >> 4) * 16
            lane = d - rbase
            sl = pl.ds(rbase, 16)
            degloc[sl] = degloc[sl] + jnp.where(iota16 == lane, wvec[l], 0.0)
        return 0

    lax.fori_loop(0, EPW // 16, chunk, 0)

    pltpu.sync_copy(degloc, stag.at[sid])
    plsc.subcore_barrier()

    @pl.when(sid < NP // _DSEG)
    def _():
        pltpu.sync_copy(stag.at[:, pl.ds(sid * _DSEG, _DSEG)], rbuf)

        def rb(v, _):
            sl = pl.ds(v * 16, 16)
            acc = rbuf[0, sl]
            for r in range(1, NS):
                acc = acc + rbuf[r, sl]
            red[sl] = acc
            return 0

        lax.fori_loop(0, _DSEG // 16, rb, 0)
        pltpu.sync_copy(red, out_hbm.at[cid, pl.ds(sid * _DSEG, _DSEG)])


_deg_kernel = functools.partial(pl.kernel, **_DEG_KW)(_deg_body)


_ZR = 25
_SCAT_KW = dict(
    out_type=jax.ShapeDtypeStruct((NC, NS, RPT, D), jnp.float32),
    mesh=_mesh,
    scratch_types=[
        pltpu.VMEM_SHARED((N, D), jnp.float32),
        pltpu.VMEM((_ZR, D), jnp.float32),
        pltpu.VMEM((EPW,), jnp.int32),
        pltpu.VMEM((EPW,), jnp.float32),
        pltpu.VMEM((CH,), jnp.int32),
        pltpu.VMEM((CH,), jnp.int32),
        pltpu.VMEM((CH,), jnp.int32),
        pltpu.VMEM((CH,), jnp.int32),
        pltpu.VMEM((CH, D), jnp.float32),
        pltpu.VMEM((CH, D), jnp.float32),
        pltpu.SemaphoreType.DMA,
        pltpu.SemaphoreType.DMA,
        pltpu.SemaphoreType.DMA,
        pltpu.SemaphoreType.DMA,
        pltpu.SemaphoreType.DMA,
        pltpu.SemaphoreType.DMA,
    ],
)


def _scatter_body(g_hbm, src_hbm, dst_hbm, ew_hbm, out_hbm,
                  accum, zbuf, srca, ewa, dstg0, dstg1, dsti0, dsti1,
                  rows0, rows1, gsem0, gsem1, dsem0, dsem1, ssem0, ssem1):
    cid = lax.axis_index("c")
    sid = lax.axis_index("s")
    wid = sid * NC + cid
    base = wid * EPW

    pltpu.async_copy(src_hbm.at[pl.ds(base, EPW)], srca, gsem0)
    pltpu.async_copy(ew_hbm.at[pl.ds(base, EPW)], ewa, gsem1)
    _zero_shared(zbuf, accum, sid, _ZR, RPT // _ZR, D, ssem0)
    pltpu.make_async_copy(src_hbm.at[pl.ds(base, EPW)], srca, gsem0).wait()
    pltpu.make_async_copy(ew_hbm.at[pl.ds(base, EPW)], ewa, gsem1).wait()
    plsc.subcore_barrier()

    def fetch(c, rows, gsem, dstg, dsem):
        pltpu.async_copy(g_hbm.at[srca.at[pl.ds(c * CH, CH)]], rows, gsem)
        pltpu.async_copy(dst_hbm.at[pl.ds(base + c * CH, CH)], dstg, dsem)

    def wait_fetch(c, rows, gsem, dstg, dsem):
        pltpu.make_async_copy(g_hbm.at[srca.at[pl.ds(c * CH, CH)]],
                              rows, gsem).wait()
        pltpu.make_async_copy(dst_hbm.at[pl.ds(base + c * CH, CH)],
                              dstg, dsem).wait()

    def scale(c, rows):
        def body(m, _):
            wvec = ewa[pl.ds(c * CH + m * 16, 16)]
            for l in range(16):
                k = m * 16 + l
                s = wvec[l]
                for j in range(D // 16):
                    sl = pl.ds(16 * j, 16)
                    rows[k, sl] = rows[k, sl] * s
            return 0

        lax.fori_loop(0, CH // 16, body, 0)

    def fill_dsti(dstg, dsti):
        for m in range(CH // 16):
            sl = pl.ds(m * 16, 16)
            dsti[sl] = dstg[sl]

    def add_start(rows, dsti, ssem):
        pltpu.async_copy(rows, accum.at[dsti], ssem, add=True)

    def add_wait(rows, dsti, ssem):
        pltpu.make_async_copy(rows, accum.at[dsti], ssem).wait()

    fetch(0, rows0, gsem0, dstg0, dsem0)

    def pair(g, _):
        c0 = 2 * g
        c1 = c0 + 1
        wait_fetch(c0, rows0, gsem0, dstg0, dsem0)

        @pl.when(g > 0)
        def _():
            add_wait(rows1, dsti1, ssem1)

        fetch(c1, rows1, gsem1, dstg1, dsem1)
        scale(c0, rows0)
        fill_dsti(dstg0, dsti0)
        add_start(rows0, dsti0, ssem0)
        wait_fetch(c1, rows1, gsem1, dstg1, dsem1)
        scale(c1, rows1)
        fill_dsti(dstg1, dsti1)
        add_wait(rows0, dsti0, ssem0)
        fetch(c1 + 1, rows0, gsem0, dstg0, dsem0)
        add_start(rows1, dsti1, ssem1)
        return 0

    lax.fori_loop(0, NCHUNK // 2, pair, 0)
    cl = NCHUNK - 1
    wait_fetch(cl, rows0, gsem0, dstg0, dsem0)
    add_wait(rows1, dsti1, ssem1)
    scale(cl, rows0)
    fill_dsti(dstg0, dsti0)
    add_start(rows0, dsti0, ssem0)
    add_wait(rows0, dsti0, ssem0)

    plsc.subcore_barrier()
    pltpu.sync_copy(accum.at[pl.ds(sid * RPT, RPT)], out_hbm.at[cid, sid])


_scatter_kernel = functools.partial(pl.kernel, **_SCAT_KW)(_scatter_body)


_BR = 400
_GRID = N // _BR


def _dinv_block(deg0, deg1):
    deg = deg0[:, :] + deg1[:, :] + 1.0
    return jnp.where(deg > 0, lax.rsqrt(jnp.maximum(deg, 1e-12)), 0.0)


def _tc1_body(deg0, deg1, x, w1, g1):
    dinv = _dinv_block(deg0, deg1)
    g1[:, :] = dinv * jnp.dot(x[:, :], w1[:, :],
                              preferred_element_type=jnp.float32)


def _tc2_body(deg0, deg1, s1a, s1b, g1, b1, w2, g2):
    dinv = _dinv_block(deg0, deg1)
    h1 = jnp.maximum(dinv * (s1a[:, :] + s1b[:, :] + g1[:, :]) + b1[:, :], 0.0)
    g2[:, :] = dinv * jnp.dot(h1, w2[:, :], preferred_element_type=jnp.float32)


def _tc3_body(deg0, deg1, s2a, s2b, g2, b2, out):
    dinv = _dinv_block(deg0, deg1)
    out[:, :] = jnp.maximum(dinv * (s2a[:, :] + s2b[:, :] + g2[:, :])
                            + b2[:, :], 0.0)


def _row_spec():
    return pl.BlockSpec((_BR, D), lambda i: (i, 0))


def _deg_spec():
    return pl.BlockSpec((_BR, 1), lambda i: (i, 0))


def _full_spec(shape):
    return pl.BlockSpec(shape, lambda i: tuple(0 for _ in shape))


def _tc1(deg0, deg1, x, w1):
    return pl.pallas_call(
        _tc1_body,
        grid=(_GRID,),
        in_specs=[_deg_spec(), _deg_spec(), _row_spec(), _full_spec((D, D))],
        out_specs=_row_spec(),
        out_shape=jax.ShapeDtypeStruct((N, D), jnp.float32),
    )(deg0, deg1, x, w1)


def _tc2(deg0, deg1, s1a, s1b, g1, b1, w2):
    return pl.pallas_call(
        _tc2_body,
        grid=(_GRID,),
        in_specs=[_deg_spec(), _deg_spec(), _row_spec(), _row_spec(),
                  _row_spec(), _full_spec((1, D)), _full_spec((D, D))],
        out_specs=_row_spec(),
        out_shape=jax.ShapeDtypeStruct((N, D), jnp.float32),
    )(deg0, deg1, s1a, s1b, g1, b1, w2)


def _tc3(deg0, deg1, s2a, s2b, g2, b2):
    return pl.pallas_call(
        _tc3_body,
        grid=(_GRID,),
        in_specs=[_deg_spec(), _deg_spec(), _row_spec(), _row_spec(),
                  _row_spec(), _full_spec((1, D))],
        out_specs=_row_spec(),
        out_shape=jax.ShapeDtypeStruct((N, D), jnp.float32),
    )(deg0, deg1, s2a, s2b, g2, b2)


def kernel(x, edge_index, edge_weight, W1, b1, W2, b2):
    src = edge_index[0].astype(jnp.int32)
    dst = edge_index[1].astype(jnp.int32)
    ew = edge_weight.astype(jnp.float32)

    degp = _deg_kernel(dst, ew)
    deg0 = degp[0, :N].reshape(N, 1)
    deg1 = degp[1, :N].reshape(N, 1)

    g1 = _tc1(deg0, deg1, x, W1)
    s1 = _scatter_kernel(g1, src, dst, ew).reshape(NC, N, D)
    g2 = _tc2(deg0, deg1, s1[0], s1[1], g1, b1.reshape(1, D), W2)
    s2 = _scatter_kernel(g2, src, dst, ew).reshape(NC, N, D)
    out = _tc3(deg0, deg1, s2[0], s2[1], g2, b2.reshape(1, D))
    return out


# speedup vs baseline: 19.0815x; 1.0542x over previous
import functools

import jax
import jax.numpy as jnp
from jax import lax
from jax.experimental import pallas as pl
from jax.experimental.pallas import tpu as pltpu
from jax.experimental.pallas import tpu_sc as plsc

N = 10000
E = 320000
D = 128
NC = 2
NS = 16
NW = NC * NS
EPW = E // NW
CH = 80
NCHUNK = EPW // CH
RPT = N // NS

_mesh = plsc.VectorSubcoreMesh(core_axis_name="c", subcore_axis_name="s")


def _zero_shared(zbuf, accum, sid, rows_per_copy, ncopies, lanes, zsem):
    nvec = lanes // 16

    def zbody(i, _):
        for j in range(nvec):
            zbuf[i, pl.ds(16 * j, 16)] = jnp.zeros((16,), jnp.float32)
        return 0

    lax.fori_loop(0, rows_per_copy, zbody, 0)
    for r in range(ncopies):
        pltpu.async_copy(zbuf, accum.at[pl.ds(sid * RPT + r * rows_per_copy,
                                              rows_per_copy)], zsem)
    for r in range(ncopies):
        pltpu.make_async_copy(zbuf, accum.at[pl.ds(sid * RPT + r * rows_per_copy,
                                                   rows_per_copy)], zsem).wait()


NP = 10240
_DSEG = NP // 8
_DEG_KW = dict(
    out_type=jax.ShapeDtypeStruct((NC, NP), jnp.float32),
    mesh=_mesh,
    scratch_types=[
        pltpu.VMEM((NP,), jnp.float32),
        pltpu.VMEM_SHARED((NS, NP), jnp.float32),
        pltpu.VMEM((NS, _DSEG), jnp.float32),
        pltpu.VMEM((_DSEG,), jnp.float32),
        pltpu.VMEM((EPW,), jnp.int32),
        pltpu.VMEM((EPW,), jnp.float32),
        pltpu.SemaphoreType.DMA,
        pltpu.SemaphoreType.DMA,
    ],
)


def _deg_body(dst_hbm, ew_hbm, out_hbm, degloc, stag, rbuf, red, dsta, ewa,
              dgsem0, dgsem1):
    cid = lax.axis_index("c")
    sid = lax.axis_index("s")
    wid = sid * NC + cid

    pltpu.async_copy(dst_hbm.at[pl.ds(wid * EPW, EPW)], dsta, dgsem0)
    pltpu.async_copy(ew_hbm.at[pl.ds(wid * EPW, EPW)], ewa, dgsem1)

    def zbody(i, _):
        degloc[pl.ds(i * 16, 16)] = jnp.zeros((16,), jnp.float32)
        return 0

    lax.fori_loop(0, NP // 16, zbody, 0)
    pltpu.make_async_copy(dst_hbm.at[pl.ds(wid * EPW, EPW)], dsta, dgsem0).wait()
    pltpu.make_async_copy(ew_hbm.at[pl.ds(wid * EPW, EPW)], ewa, dgsem1).wait()

    iota16 = lax.iota(jnp.int32, 16)

    def chunk(m, _):
        dvec = dsta[pl.ds(m * 16, 16)]
        wvec = ewa[pl.ds(m * 16, 16)]
        for l in range(16):
            d = dvec[l]
            rbase = (d >> 4) * 16
            lane = d - rbase
            sl = pl.ds(rbase, 16)
            degloc[sl] = degloc[sl] + jnp.where(iota16 == lane, wvec[l], 0.0)
        return 0

    lax.fori_loop(0, EPW // 16, chunk, 0)

    pltpu.sync_copy(degloc, stag.at[sid])
    plsc.subcore_barrier()

    @pl.when(sid < NP // _DSEG)
    def _():
        pltpu.sync_copy(stag.at[:, pl.ds(sid * _DSEG, _DSEG)], rbuf)

        def rb(v, _):
            sl = pl.ds(v * 16, 16)
            acc = rbuf[0, sl]
            for r in range(1, NS):
                acc = acc + rbuf[r, sl]
            red[sl] = acc
            return 0

        lax.fori_loop(0, _DSEG // 16, rb, 0)
        pltpu.sync_copy(red, out_hbm.at[cid, pl.ds(sid * _DSEG, _DSEG)])


_deg_kernel = functools.partial(pl.kernel, **_DEG_KW)(_deg_body)


_ZR = 25
_SCAT_KW = dict(
    out_type=jax.ShapeDtypeStruct((NC, NS, RPT, D), jnp.float32),
    mesh=_mesh,
    scratch_types=[
        pltpu.VMEM_SHARED((N, D), jnp.float32),
        pltpu.VMEM((_ZR, D), jnp.float32),
        pltpu.VMEM((EPW,), jnp.int32),
        pltpu.VMEM((EPW,), jnp.float32),
        pltpu.VMEM((CH,), jnp.int32),
        pltpu.VMEM((CH,), jnp.int32),
        pltpu.VMEM((CH,), jnp.int32),
        pltpu.VMEM((CH,), jnp.int32),
        pltpu.VMEM((CH, D), jnp.float32),
        pltpu.VMEM((CH, D), jnp.float32),
        pltpu.SemaphoreType.DMA,
        pltpu.SemaphoreType.DMA,
        pltpu.SemaphoreType.DMA,
        pltpu.SemaphoreType.DMA,
        pltpu.SemaphoreType.DMA,
        pltpu.SemaphoreType.DMA,
    ],
)


def _scatter_body(g_hbm, src_hbm, dst_hbm, ew_hbm, out_hbm,
                  accum, zbuf, srca, ewa, dstg0, dstg1, dsti0, dsti1,
                  rows0, rows1, gsem0, gsem1, dsem0, dsem1, ssem0, ssem1):
    cid = lax.axis_index("c")
    sid = lax.axis_index("s")
    wid = sid * NC + cid
    base = wid * EPW

    pltpu.async_copy(src_hbm.at[pl.ds(base, EPW)], srca, gsem0)
    pltpu.async_copy(ew_hbm.at[pl.ds(base, EPW)], ewa, gsem1)
    _zero_shared(zbuf, accum, sid, _ZR, RPT // _ZR, D, ssem0)
    pltpu.make_async_copy(src_hbm.at[pl.ds(base, EPW)], srca, gsem0).wait()
    pltpu.make_async_copy(ew_hbm.at[pl.ds(base, EPW)], ewa, gsem1).wait()
    plsc.subcore_barrier()

    def fetch(c, rows, gsem, dstg, dsem):
        pltpu.async_copy(g_hbm.at[srca.at[pl.ds(c * CH, CH)]], rows, gsem)
        pltpu.async_copy(dst_hbm.at[pl.ds(base + c * CH, CH)], dstg, dsem)

    def wait_fetch(c, rows, gsem, dstg, dsem):
        pltpu.make_async_copy(g_hbm.at[srca.at[pl.ds(c * CH, CH)]],
                              rows, gsem).wait()
        pltpu.make_async_copy(dst_hbm.at[pl.ds(base + c * CH, CH)],
                              dstg, dsem).wait()

    def scale(c, rows):
        def body(m, _):
            wvec = ewa[pl.ds(c * CH + m * 16, 16)]
            for l in range(16):
                k = m * 16 + l
                s = wvec[l]
                for j in range(D // 16):
                    sl = pl.ds(16 * j, 16)
                    rows[k, sl] = rows[k, sl] * s
            return 0

        lax.fori_loop(0, CH // 16, body, 0)

    def fill_dsti(dstg, dsti):
        for m in range(CH // 16):
            sl = pl.ds(m * 16, 16)
            dsti[sl] = dstg[sl]

    def add_start(rows, dsti, ssem):
        pltpu.async_copy(rows, accum.at[dsti], ssem, add=True)

    def add_wait(rows, dsti, ssem):
        pltpu.make_async_copy(rows, accum.at[dsti], ssem).wait()

    fetch(0, rows0, gsem0, dstg0, dsem0)

    def pair(g, _):
        c0 = 2 * g
        c1 = c0 + 1
        wait_fetch(c0, rows0, gsem0, dstg0, dsem0)

        @pl.when(g > 0)
        def _():
            add_wait(rows1, dsti1, ssem1)

        fetch(c1, rows1, gsem1, dstg1, dsem1)
        scale(c0, rows0)
        fill_dsti(dstg0, dsti0)
        add_start(rows0, dsti0, ssem0)
        wait_fetch(c1, rows1, gsem1, dstg1, dsem1)
        scale(c1, rows1)
        fill_dsti(dstg1, dsti1)
        add_wait(rows0, dsti0, ssem0)
        fetch(c1 + 1, rows0, gsem0, dstg0, dsem0)
        add_start(rows1, dsti1, ssem1)
        return 0

    lax.fori_loop(0, NCHUNK // 2, pair, 0)
    cl = NCHUNK - 1
    wait_fetch(cl, rows0, gsem0, dstg0, dsem0)
    add_wait(rows1, dsti1, ssem1)
    scale(cl, rows0)
    fill_dsti(dstg0, dsti0)
    add_start(rows0, dsti0, ssem0)
    add_wait(rows0, dsti0, ssem0)

    plsc.subcore_barrier()
    pltpu.sync_copy(accum.at[pl.ds(sid * RPT, RPT)], out_hbm.at[cid, sid])


_scatter_kernel = functools.partial(pl.kernel, **_SCAT_KW)(_scatter_body)


_BR = 2000
_GRID = N // _BR


def _dinv_block(deg0, deg1):
    deg = deg0[:, :] + deg1[:, :] + 1.0
    return jnp.where(deg > 0, lax.rsqrt(jnp.maximum(deg, 1e-12)), 0.0)


def _tc1_body(deg0, deg1, x, w1, g1):
    dinv = _dinv_block(deg0, deg1)
    g1[:, :] = dinv * jnp.dot(x[:, :], w1[:, :],
                              preferred_element_type=jnp.float32)


def _tc2_body(deg0, deg1, s1a, s1b, g1, b1, w2, g2):
    dinv = _dinv_block(deg0, deg1)
    h1 = jnp.maximum(dinv * (s1a[:, :] + s1b[:, :] + g1[:, :]) + b1[:, :], 0.0)
    g2[:, :] = dinv * jnp.dot(h1, w2[:, :], preferred_element_type=jnp.float32)


def _tc3_body(deg0, deg1, s2a, s2b, g2, b2, out):
    dinv = _dinv_block(deg0, deg1)
    out[:, :] = jnp.maximum(dinv * (s2a[:, :] + s2b[:, :] + g2[:, :])
                            + b2[:, :], 0.0)


def _row_spec():
    return pl.BlockSpec((_BR, D), lambda i: (i, 0))


def _deg_spec():
    return pl.BlockSpec((_BR, 1), lambda i: (i, 0))


def _full_spec(shape):
    return pl.BlockSpec(shape, lambda i: tuple(0 for _ in shape))


def _tc1(deg0, deg1, x, w1):
    return pl.pallas_call(
        _tc1_body,
        grid=(_GRID,),
        in_specs=[_deg_spec(), _deg_spec(), _row_spec(), _full_spec((D, D))],
        out_specs=_row_spec(),
        out_shape=jax.ShapeDtypeStruct((N, D), jnp.float32),
    )(deg0, deg1, x, w1)


def _tc2(deg0, deg1, s1a, s1b, g1, b1, w2):
    return pl.pallas_call(
        _tc2_body,
        grid=(_GRID,),
        in_specs=[_deg_spec(), _deg_spec(), _row_spec(), _row_spec(),
                  _row_spec(), _full_spec((1, D)), _full_spec((D, D))],
        out_specs=_row_spec(),
        out_shape=jax.ShapeDtypeStruct((N, D), jnp.float32),
    )(deg0, deg1, s1a, s1b, g1, b1, w2)


def _tc3(deg0, deg1, s2a, s2b, g2, b2):
    return pl.pallas_call(
        _tc3_body,
        grid=(_GRID,),
        in_specs=[_deg_spec(), _deg_spec(), _row_spec(), _row_spec(),
                  _row_spec(), _full_spec((1, D))],
        out_specs=_row_spec(),
        out_shape=jax.ShapeDtypeStruct((N, D), jnp.float32),
    )(deg0, deg1, s2a, s2b, g2, b2)


def kernel(x, edge_index, edge_weight, W1, b1, W2, b2):
    src = edge_index[0].astype(jnp.int32)
    dst = edge_index[1].astype(jnp.int32)
    ew = edge_weight.astype(jnp.float32)

    degp = _deg_kernel(dst, ew)
    deg0 = degp[0, :N].reshape(N, 1)
    deg1 = degp[1, :N].reshape(N, 1)

    g1 = _tc1(deg0, deg1, x, W1)
    s1 = _scatter_kernel(g1, src, dst, ew).reshape(NC, N, D)
    g2 = _tc2(deg0, deg1, s1[0], s1[1], g1, b1.reshape(1, D), W2)
    s2 = _scatter_kernel(g2, src, dst, ew).reshape(NC, N, D)
    out = _tc3(deg0, deg1, s2[0], s2[1], g2, b2.reshape(1, D))
    return out
